# Initial kernel scaffold; baseline (speedup 1.0000x reference)
import jax
import jax.numpy as jnp
from jax.experimental import pallas as pl


def kernel(x, edge_index, W_neigh, bias):
    raise NotImplementedError("write your pallas kernel here")


# speedup vs baseline: 7.7102x; 7.7102x over previous
import functools

import jax
import jax.numpy as jnp
from jax import lax
from jax.experimental import pallas as pl
from jax.experimental.pallas import tpu as pltpu
from jax.experimental.pallas import tpu_sc as plsc

N_NODES = 10000
N_PAD = 10240
N_EDGES = 320000
D = 128
DA = 144

NC = 2
NS = 16
NW = NC * NS
EPW = N_EDGES // NW
CHUNK = 80
NCHUNK = EPW // CHUNK
RPS = N_PAD // NS


def _sc_aggregate(x_aug, src3d, dst3d):
    mesh = plsc.VectorSubcoreMesh(core_axis_name="c", subcore_axis_name="s")

    @functools.partial(
        pl.kernel,
        mesh=mesh,
        out_type=jax.ShapeDtypeStruct((NC, N_PAD, DA), jnp.float32),
        scratch_types=[
            pltpu.VMEM((NCHUNK, CHUNK), jnp.int32),
            pltpu.VMEM((NCHUNK, CHUNK), jnp.int32),
            pltpu.VMEM((CHUNK, DA), jnp.float32),
            pltpu.VMEM_SHARED((N_PAD, DA), jnp.float32),
            pltpu.SemaphoreType.DMA,
        ],
        compiler_params=pltpu.CompilerParams(use_tc_tiling_on_sc=False),
    )
    def body(x_hbm, src_hbm, dst_hbm, agg_out, src_v, dst_v, rows_v,
             agg_sh, sem):
        c = lax.axis_index("c")
        s = lax.axis_index("s")
        wid = c * NS + s

        def zero_rows(i, carry):
            for k in range(DA // 16):
                rows_v[i, pl.ds(k * 16, 16)] = jnp.zeros((16,), jnp.float32)
            return carry
        lax.fori_loop(0, CHUNK, zero_rows, 0)
        for b in range(RPS // CHUNK):
            pltpu.sync_copy(rows_v,
                            agg_sh.at[pl.ds(s * RPS + b * CHUNK, CHUNK)])

        pltpu.sync_copy(src_hbm.at[wid], src_v)
        pltpu.sync_copy(dst_hbm.at[wid], dst_v)

        plsc.subcore_barrier()

        def edge_chunk(j, carry):
            pltpu.async_copy(x_hbm.at[src_v.at[j]], rows_v, sem).wait()
            pltpu.sync_copy(rows_v, agg_sh.at[dst_v.at[j]], add=True)
            return carry
        lax.fori_loop(0, NCHUNK, edge_chunk, 0)

        plsc.subcore_barrier()

        pltpu.sync_copy(agg_sh.at[pl.ds(s * RPS, RPS)],
                        agg_out.at[c, pl.ds(s * RPS, RPS)])

    return body(x_aug, src3d, dst3d)


def _tc_body(agg_ref, x_ref, w_ref, b_ref, out_ref):
    a0 = agg_ref[0]
    a1 = agg_ref[1]
    aggs = a0[:, :D] + a1[:, :D] + x_ref[...]
    deg = a0[:, D:D + 1] + a1[:, D:D + 1] + 1.0
    h = aggs / deg
    out_ref[...] = (
        jnp.dot(h, w_ref[...], preferred_element_type=jnp.float32) + b_ref[...]
    )


def _tc_finish(agg_part, x, w, bias2d):
    blk = 1000
    grid = (N_NODES // blk,)
    return pl.pallas_call(
        _tc_body,
        grid=grid,
        in_specs=[
            pl.BlockSpec((NC, blk, DA), lambda i: (0, i, 0)),
            pl.BlockSpec((blk, D), lambda i: (i, 0)),
            pl.BlockSpec((D, D), lambda i: (0, 0)),
            pl.BlockSpec((1, D), lambda i: (0, 0)),
        ],
        out_specs=pl.BlockSpec((blk, D), lambda i: (i, 0)),
        out_shape=jax.ShapeDtypeStruct((N_NODES, D), jnp.float32),
    )(agg_part, x, w, bias2d)


def kernel(x, edge_index, W_neigh, bias):
    x_aug = jnp.concatenate(
        [x, jnp.ones((N_NODES, DA - D), jnp.float32)], axis=1)
    src3d = edge_index[0].reshape(NW, NCHUNK, CHUNK)
    dst3d = edge_index[1].reshape(NW, NCHUNK, CHUNK)
    agg_part = _sc_aggregate(x_aug, src3d, dst3d)
    return _tc_finish(agg_part, x, W_neigh, bias.reshape(1, D))


# speedup vs baseline: 9.0451x; 1.1731x over previous
import functools

import jax
import jax.numpy as jnp
from jax import lax
from jax.experimental import pallas as pl
from jax.experimental.pallas import tpu as pltpu
from jax.experimental.pallas import tpu_sc as plsc

N_NODES = 10000
N_PAD = 10240
N_EDGES = 320000
D = 128
DA = 144

NC = 2
NS = 16
NW = NC * NS
EPW = N_EDGES // NW
CHUNK = 100
NCHUNK = EPW // CHUNK
CPP = 20
NPHASE = NCHUNK // CPP
RPS = N_PAD // NS
ZROWS = 128


def _sc_aggregate(x_aug, src3d, dst3d):
    mesh = plsc.VectorSubcoreMesh(core_axis_name="c", subcore_axis_name="s")

    @functools.partial(
        pl.kernel,
        mesh=mesh,
        out_type=jax.ShapeDtypeStruct((NC, N_PAD, DA), jnp.float32),
        scratch_types=[
            pltpu.VMEM((CPP, CHUNK), jnp.int32),
            pltpu.VMEM((CPP, CHUNK), jnp.int32),
            pltpu.VMEM((CHUNK, DA), jnp.float32),
            pltpu.VMEM((CHUNK, DA), jnp.float32),
            pltpu.VMEM_SHARED((N_PAD, DA), jnp.float32),
            pltpu.SemaphoreType.DMA,
            pltpu.SemaphoreType.DMA,
        ],
        compiler_params=pltpu.CompilerParams(use_tc_tiling_on_sc=False),
    )
    def body(x_hbm, src_hbm, dst_hbm, agg_out, src_v, dst_v, rows_a, rows_b,
             agg_sh, sem_a, sem_b):
        c = lax.axis_index("c")
        s = lax.axis_index("s")
        wid = c * NS + s

        def zero_rows(i, carry):
            for k in range(DA // 16):
                rows_a[i, pl.ds(k * 16, 16)] = jnp.zeros((16,), jnp.float32)
            return carry
        lax.fori_loop(0, CHUNK, zero_rows, 0)
        for b in range(RPS // 80):
            pltpu.sync_copy(rows_a.at[pl.ds(0, 80)],
                            agg_sh.at[pl.ds(s * RPS + b * 80, 80)])

        plsc.subcore_barrier()

        def gwait(buf, sem):
            pltpu.make_async_copy(x_hbm.at[pl.ds(0, CHUNK)], buf, sem).wait()

        def phase(p, carry):
            pltpu.sync_copy(src_hbm.at[wid, pl.ds(p * CPP, CPP)], src_v)
            pltpu.sync_copy(dst_hbm.at[wid, pl.ds(p * CPP, CPP)], dst_v)
            pltpu.async_copy(x_hbm.at[src_v.at[0]], rows_a, sem_a)

            def pair(g, carry2):
                j0 = 2 * g
                gwait(rows_a, sem_a)
                pltpu.async_copy(x_hbm.at[src_v.at[j0 + 1]], rows_b, sem_b)
                pltpu.sync_copy(rows_a, agg_sh.at[dst_v.at[j0]], add=True)
                gwait(rows_b, sem_b)
                pltpu.async_copy(x_hbm.at[src_v.at[j0 + 2]], rows_a, sem_a)
                pltpu.sync_copy(rows_b, agg_sh.at[dst_v.at[j0 + 1]],
                                add=True)
                return carry2
            lax.fori_loop(0, CPP // 2 - 1, pair, 0)

            j0 = CPP - 2
            gwait(rows_a, sem_a)
            pltpu.async_copy(x_hbm.at[src_v.at[j0 + 1]], rows_b, sem_b)
            pltpu.sync_copy(rows_a, agg_sh.at[dst_v.at[j0]], add=True)
            gwait(rows_b, sem_b)
            pltpu.sync_copy(rows_b, agg_sh.at[dst_v.at[j0 + 1]], add=True)
            return carry
        lax.fori_loop(0, NPHASE, phase, 0)

        plsc.subcore_barrier()

        pltpu.sync_copy(agg_sh.at[pl.ds(s * RPS, RPS)],
                        agg_out.at[c, pl.ds(s * RPS, RPS)])

    return body(x_aug, src3d, dst3d)


def _tc_body(agg_ref, x_ref, w_ref, b_ref, out_ref):
    a0 = agg_ref[0]
    a1 = agg_ref[1]
    aggs = a0[:, :D] + a1[:, :D] + x_ref[...]
    deg = a0[:, D:D + 1] + a1[:, D:D + 1] + 1.0
    h = aggs / deg
    out_ref[...] = (
        jnp.dot(h, w_ref[...], preferred_element_type=jnp.float32) + b_ref[...]
    )


def _tc_finish(agg_part, x, w, bias2d):
    blk = 1000
    grid = (N_NODES // blk,)
    return pl.pallas_call(
        _tc_body,
        grid=grid,
        in_specs=[
            pl.BlockSpec((NC, blk, DA), lambda i: (0, i, 0)),
            pl.BlockSpec((blk, D), lambda i: (i, 0)),
            pl.BlockSpec((D, D), lambda i: (0, 0)),
            pl.BlockSpec((1, D), lambda i: (0, 0)),
        ],
        out_specs=pl.BlockSpec((blk, D), lambda i: (i, 0)),
        out_shape=jax.ShapeDtypeStruct((N_NODES, D), jnp.float32),
    )(agg_part, x, w, bias2d)


def kernel(x, edge_index, W_neigh, bias):
    x_aug = jnp.concatenate(
        [x, jnp.ones((N_NODES, DA - D), jnp.float32)], axis=1)
    src3d = edge_index[0].reshape(NW, NCHUNK, CHUNK)
    dst3d = edge_index[1].reshape(NW, NCHUNK, CHUNK)
    agg_part = _sc_aggregate(x_aug, src3d, dst3d)
    return _tc_finish(agg_part, x, W_neigh, bias.reshape(1, D))


# speedup vs baseline: 9.1166x; 1.0079x over previous
import functools

import jax
import jax.numpy as jnp
from jax import lax
from jax.experimental import pallas as pl
from jax.experimental.pallas import tpu as pltpu
from jax.experimental.pallas import tpu_sc as plsc

N_NODES = 10000
N_PAD = 10240
N_EDGES = 320000
D = 128
DA = 144

NC = 2
NS = 16
NW = NC * NS
EPW = N_EDGES // NW
CHUNK = 100
NCHUNK = EPW // CHUNK
CPP = 20
NPHASE = NCHUNK // CPP
RPS = N_PAD // NS
ZROWS = 128


def _sc_aggregate(x_aug, src3d, dst3d):
    mesh = plsc.VectorSubcoreMesh(core_axis_name="c", subcore_axis_name="s")

    @functools.partial(
        pl.kernel,
        mesh=mesh,
        out_type=jax.ShapeDtypeStruct((NC, N_PAD, DA), jnp.float32),
        scratch_types=[
            pltpu.VMEM((CPP, CHUNK), jnp.int32),
            pltpu.VMEM((CPP, CHUNK), jnp.int32),
            pltpu.VMEM((CHUNK, DA), jnp.float32),
            pltpu.VMEM((CHUNK, DA), jnp.float32),
            pltpu.VMEM_SHARED((N_PAD, DA), jnp.float32),
            pltpu.SemaphoreType.DMA,
            pltpu.SemaphoreType.DMA,
        ],
        compiler_params=pltpu.CompilerParams(use_tc_tiling_on_sc=False),
    )
    def body(x_hbm, src_hbm, dst_hbm, agg_out, src_v, dst_v, rows_a, rows_b,
             agg_sh, sem_a, sem_b):
        c = lax.axis_index("c")
        s = lax.axis_index("s")
        wid = c * NS + s

        def zero_rows(i, carry):
            for k in range(DA // 16):
                rows_a[i, pl.ds(k * 16, 16)] = jnp.zeros((16,), jnp.float32)
            return carry
        lax.fori_loop(0, CHUNK, zero_rows, 0)
        for b in range(RPS // 80):
            pltpu.sync_copy(rows_a.at[pl.ds(0, 80)],
                            agg_sh.at[pl.ds(s * RPS + b * 80, 80)])

        plsc.subcore_barrier()

        def gwait(buf, sem):
            pltpu.make_async_copy(x_hbm.at[pl.ds(0, CHUNK)], buf, sem).wait()

        def phase(p, carry):
            pltpu.sync_copy(src_hbm.at[wid, pl.ds(p * CPP, CPP)], src_v)
            pltpu.sync_copy(dst_hbm.at[wid, pl.ds(p * CPP, CPP)], dst_v)
            pltpu.async_copy(x_hbm.at[src_v.at[0]], rows_a, sem_a)

            def pair(g, carry2):
                j0 = 2 * g
                gwait(rows_a, sem_a)
                pltpu.async_copy(x_hbm.at[src_v.at[j0 + 1]], rows_b, sem_b)
                pltpu.sync_copy(rows_a, agg_sh.at[dst_v.at[j0]], add=True)
                gwait(rows_b, sem_b)
                pltpu.async_copy(x_hbm.at[src_v.at[j0 + 2]], rows_a, sem_a)
                pltpu.sync_copy(rows_b, agg_sh.at[dst_v.at[j0 + 1]],
                                add=True)
                return carry2
            lax.fori_loop(0, CPP // 2 - 1, pair, 0)

            j0 = CPP - 2
            gwait(rows_a, sem_a)
            pltpu.async_copy(x_hbm.at[src_v.at[j0 + 1]], rows_b, sem_b)
            pltpu.sync_copy(rows_a, agg_sh.at[dst_v.at[j0]], add=True)
            gwait(rows_b, sem_b)
            pltpu.sync_copy(rows_b, agg_sh.at[dst_v.at[j0 + 1]], add=True)
            return carry
        lax.fori_loop(0, NPHASE, phase, 0)

        plsc.subcore_barrier()

        pltpu.sync_copy(agg_sh.at[pl.ds(s * RPS, RPS)],
                        agg_out.at[c, pl.ds(s * RPS, RPS)])

    return body(x_aug, src3d, dst3d)


def _tc_body(agg_ref, x_ref, w_ref, b_ref, out_ref):
    a0 = agg_ref[0]
    a1 = agg_ref[1]
    aggs = a0[:, :D] + a1[:, :D] + x_ref[...]
    deg = a0[:, D:D + 1] + a1[:, D:D + 1] + 1.0
    h = aggs / deg
    out_ref[...] = (
        jnp.dot(h, w_ref[...], preferred_element_type=jnp.float32) + b_ref[...]
    )


def _tc_finish(agg_part, x, w, bias2d):
    blk = 2000
    grid = (N_NODES // blk,)
    return pl.pallas_call(
        _tc_body,
        grid=grid,
        in_specs=[
            pl.BlockSpec((NC, blk, DA), lambda i: (0, i, 0)),
            pl.BlockSpec((blk, D), lambda i: (i, 0)),
            pl.BlockSpec((D, D), lambda i: (0, 0)),
            pl.BlockSpec((1, D), lambda i: (0, 0)),
        ],
        out_specs=pl.BlockSpec((blk, D), lambda i: (i, 0)),
        out_shape=jax.ShapeDtypeStruct((N_NODES, D), jnp.float32),
    )(agg_part, x, w, bias2d)


def kernel(x, edge_index, W_neigh, bias):
    x_aug = jnp.concatenate(
        [x, jnp.ones((N_NODES, DA - D), jnp.float32)], axis=1)
    src3d = edge_index[0].reshape(NW, NCHUNK, CHUNK)
    dst3d = edge_index[1].reshape(NW, NCHUNK, CHUNK)
    agg_part = _sc_aggregate(x_aug, src3d, dst3d)
    return _tc_finish(agg_part, x, W_neigh, bias.reshape(1, D))
